# Initial kernel scaffold; baseline (speedup 1.0000x reference)
import jax
import jax.numpy as jnp
from jax.experimental import pallas as pl


def kernel(features, coordinates, dis_mats, W1, b1, g1, be1, W2, b2, g2, be2):
    raise NotImplementedError("write your pallas kernel here")


# speedup vs baseline: 4.5977x; 4.5977x over previous
import functools

import jax
import jax.numpy as jnp
from jax.experimental import pallas as pl
from jax.experimental.pallas import tpu as pltpu

_R2 = 0.18 ** 2
_K = 32
_EPS = 1e-5
_N = 1024
_ST = 256

_PREC = jax.lax.Precision.HIGHEST


def _ln(h, g, b):
    mu = jnp.mean(h, axis=-1, keepdims=True)
    var = jnp.mean((h - mu) ** 2, axis=-1, keepdims=True)
    return (h - mu) * jax.lax.rsqrt(var + _EPS) * g + b


def _k1_body(f_ref, w1_ref, b1_ref, g1_ref, be1_ref, wzy_ref, bzy_ref, zy_ref):
    h = jnp.dot(f_ref[0], w1_ref[...], preferred_element_type=jnp.float32,
                precision=_PREC)
    h = h + b1_ref[...][None, :]
    x = jnp.maximum(_ln(h, g1_ref[...][None, :], be1_ref[...][None, :]), 0.0)
    zy = jnp.dot(x, wzy_ref[...], preferred_element_type=jnp.float32,
                 precision=_PREC)
    zy_ref[0] = zy + bzy_ref[...][None, :]


def _cumsum_lanes(x, n):
    d = 1
    while d < n:
        shifted = jnp.concatenate(
            [jnp.zeros(x.shape[:-1] + (d,), x.dtype), x[..., : n - d]], axis=-1)
        x = x + shifted
        d *= 2
    return x


def _k23_body(dis_ref, z_ref, y_ref, g2_ref, be2_ref, out_ref):
    dis = dis_ref[0]
    mask = (dis <= _R2).astype(jnp.float32)
    rank = _cumsum_lanes(mask, _N)

    y = y_ref[0]
    z = z_ref[0]
    g2 = g2_ref[...][None, :]
    be2 = be2_ref[...][None, :]

    lanes = jax.lax.broadcasted_iota(jnp.int32, (_ST, _N), 1)

    idx1 = jnp.sum((rank < 1.0).astype(jnp.float32), axis=-1, keepdims=True)

    acc_s = jnp.zeros((_ST, 128), jnp.float32)
    acc_m = jnp.full((_ST, 128), -jnp.inf, jnp.float32)
    for k in range(1, _K + 1):
        if k == 1:
            idxk = idx1
        else:
            idxk = jnp.sum((rank < float(k)).astype(jnp.float32), axis=-1,
                           keepdims=True)
            idxk = jnp.where(idxk == float(_N), idx1, idxk)
        idxk = jnp.minimum(idxk, float(_N - 1))
        onehot = (lanes == idxk.astype(jnp.int32)).astype(jnp.float32)
        w = jnp.dot(onehot, y, preferred_element_type=jnp.float32,
                    precision=_PREC)
        v = jnp.maximum(_ln(z - w, g2, be2), 0.0)
        acc_s = acc_s + v
        acc_m = jnp.maximum(acc_m, v)

    out_ref[0] = jnp.concatenate([acc_s * (1.0 / _K), acc_m], axis=-1)


@jax.jit
def kernel(features, coordinates, dis_mats, W1, b1, g1, be1, W2, b2, g2, be2):
    del coordinates
    B, N, init_dim = features.shape
    dim = W1.shape[1]
    half = W2.shape[1]

    w2a, w2b = W2[:dim], W2[dim:]
    wzy = jnp.concatenate([w2a + w2b, w2b], axis=1)
    bzy = jnp.concatenate([b2, jnp.zeros((half,), b2.dtype)])

    zy = pl.pallas_call(
        _k1_body,
        grid=(B,),
        in_specs=[
            pl.BlockSpec((1, N, init_dim), lambda b: (b, 0, 0)),
            pl.BlockSpec((init_dim, dim), lambda b: (0, 0)),
            pl.BlockSpec((dim,), lambda b: (0,)),
            pl.BlockSpec((dim,), lambda b: (0,)),
            pl.BlockSpec((dim,), lambda b: (0,)),
            pl.BlockSpec((dim, 2 * half), lambda b: (0, 0)),
            pl.BlockSpec((2 * half,), lambda b: (0,)),
        ],
        out_specs=pl.BlockSpec((1, N, 2 * half), lambda b: (b, 0, 0)),
        out_shape=jax.ShapeDtypeStruct((B, N, 2 * half), jnp.float32),
        compiler_params=pltpu.CompilerParams(
            dimension_semantics=("parallel",)),
    )(features, W1, b1, g1, be1, wzy, bzy)

    z = zy[..., :half]
    y = zy[..., half:]

    out = pl.pallas_call(
        _k23_body,
        grid=(B, N // _ST),
        in_specs=[
            pl.BlockSpec((1, _ST, N), lambda b, t: (b, t, 0)),
            pl.BlockSpec((1, _ST, half), lambda b, t: (b, t, 0)),
            pl.BlockSpec((1, N, half), lambda b, t: (b, 0, 0)),
            pl.BlockSpec((half,), lambda b, t: (0,)),
            pl.BlockSpec((half,), lambda b, t: (0,)),
        ],
        out_specs=pl.BlockSpec((1, _ST, 2 * half), lambda b, t: (b, t, 0)),
        out_shape=jax.ShapeDtypeStruct((B, N, 2 * half), jnp.float32),
        compiler_params=pltpu.CompilerParams(
            dimension_semantics=("parallel", "parallel")),
    )(dis_mats, z, y, g2, be2)

    return out


# speedup vs baseline: 11.6624x; 2.5366x over previous
import functools

import jax
import jax.numpy as jnp
from jax.experimental import pallas as pl
from jax.experimental.pallas import tpu as pltpu

_R2 = 0.18 ** 2
_K = 32
_EPS = 1e-5
_N = 1024
_ST = 128

_PREC = jax.lax.Precision.HIGHEST


def _ln(h, g, b):
    mu = jnp.mean(h, axis=-1, keepdims=True)
    var = jnp.mean((h - mu) ** 2, axis=-1, keepdims=True)
    return (h - mu) * jax.lax.rsqrt(var + _EPS) * g + b


def _k1_body(f_ref, w1_ref, b1_ref, g1_ref, be1_ref, wzy_ref, bzy_ref, zy_ref):
    h = jnp.dot(f_ref[0], w1_ref[...], preferred_element_type=jnp.float32,
                precision=_PREC)
    h = h + b1_ref[...][None, :]
    x = jnp.maximum(_ln(h, g1_ref[...][None, :], be1_ref[...][None, :]), 0.0)
    zy = jnp.dot(x, wzy_ref[...], preferred_element_type=jnp.float32,
                 precision=_PREC)
    zy_ref[0] = zy + bzy_ref[...][None, :]


def _k23_body(dis_ref, tri_ref, z_ref, y_ref, g2_ref, be2_ref, out_ref):
    dis = dis_ref[0]
    mask = dis <= _R2
    rank = jnp.dot(mask.astype(jnp.bfloat16), tri_ref[...],
                   preferred_element_type=jnp.float32)
    count = rank[:, _N - 1:]

    empty = count == 0.0
    rankm = jnp.where(mask, rank, -1.0)
    lastlane = jax.lax.broadcasted_iota(jnp.int32, (1, _N), 1) == _N - 1
    rankm = jnp.where(empty & lastlane, 0.0, rankm)

    y = y_ref[0]
    yh = y.astype(jnp.bfloat16)
    yl = (y - yh.astype(jnp.float32)).astype(jnp.bfloat16)

    z = z_ref[0]
    g2 = g2_ref[...][None, :]
    be2 = be2_ref[...][None, :]

    acc_s = jnp.zeros((_ST, 128), jnp.float32)
    acc_m = jnp.full((_ST, 128), -jnp.inf, jnp.float32)
    for k in range(1, _K + 1):
        selk = jnp.where(count >= float(k), float(k), 1.0)
        selk = jnp.where(empty, 0.0, selk)
        ohk = (rankm == selk).astype(jnp.bfloat16)
        w = (jnp.dot(ohk, yh, preferred_element_type=jnp.float32)
             + jnp.dot(ohk, yl, preferred_element_type=jnp.float32))
        v = jnp.maximum(_ln(z - w, g2, be2), 0.0)
        acc_s = acc_s + v
        acc_m = jnp.maximum(acc_m, v)

    out_ref[0] = jnp.concatenate([acc_s * (1.0 / _K), acc_m], axis=-1)


@jax.jit
def kernel(features, coordinates, dis_mats, W1, b1, g1, be1, W2, b2, g2, be2):
    del coordinates
    B, N, init_dim = features.shape
    dim = W1.shape[1]
    half = W2.shape[1]

    w2a, w2b = W2[:dim], W2[dim:]
    wzy = jnp.concatenate([w2a + w2b, w2b], axis=1)
    bzy = jnp.concatenate([b2, jnp.zeros((half,), b2.dtype)])

    zy = pl.pallas_call(
        _k1_body,
        grid=(B,),
        in_specs=[
            pl.BlockSpec((1, N, init_dim), lambda b: (b, 0, 0)),
            pl.BlockSpec((init_dim, dim), lambda b: (0, 0)),
            pl.BlockSpec((dim,), lambda b: (0,)),
            pl.BlockSpec((dim,), lambda b: (0,)),
            pl.BlockSpec((dim,), lambda b: (0,)),
            pl.BlockSpec((dim, 2 * half), lambda b: (0, 0)),
            pl.BlockSpec((2 * half,), lambda b: (0,)),
        ],
        out_specs=pl.BlockSpec((1, N, 2 * half), lambda b: (b, 0, 0)),
        out_shape=jax.ShapeDtypeStruct((B, N, 2 * half), jnp.float32),
        compiler_params=pltpu.CompilerParams(
            dimension_semantics=("parallel",)),
    )(features, W1, b1, g1, be1, wzy, bzy)

    z = zy[..., :half]
    y = zy[..., half:]

    tri = (jnp.arange(N)[:, None] <= jnp.arange(N)[None, :]).astype(
        jnp.bfloat16)

    out = pl.pallas_call(
        _k23_body,
        grid=(B, N // _ST),
        in_specs=[
            pl.BlockSpec((1, _ST, N), lambda b, t: (b, t, 0)),
            pl.BlockSpec((N, N), lambda b, t: (0, 0)),
            pl.BlockSpec((1, _ST, half), lambda b, t: (b, t, 0)),
            pl.BlockSpec((1, N, half), lambda b, t: (b, 0, 0)),
            pl.BlockSpec((half,), lambda b, t: (0,)),
            pl.BlockSpec((half,), lambda b, t: (0,)),
        ],
        out_specs=pl.BlockSpec((1, _ST, 2 * half), lambda b, t: (b, t, 0)),
        out_shape=jax.ShapeDtypeStruct((B, N, 2 * half), jnp.float32),
        compiler_params=pltpu.CompilerParams(
            dimension_semantics=("parallel", "parallel")),
    )(dis_mats, tri, z, y, g2, be2)

    return out
